# baseline (speedup 1.0000x reference)
import functools

import jax
import jax.numpy as jnp
from jax import lax
from jax.experimental import pallas as pl
from jax.experimental.pallas import tpu as pltpu
from jax.experimental.pallas import tpu_sc as plsc

B = 16384
OUT_D = 128
NAME_ROWS = 10
TYPE_ROWS = 8
EMB_D = 5
N_COMB = NAME_ROWS * TYPE_ROWS

_NC = 2
_NS = 16
_NW = _NC * _NS
_BPW = B // _NW
_CHUNK = 128
_NCHUNK = _BPW // _CHUNK


def _tc_body(feat_ref, name_ref, type_ref, w_ref, b_ref, table_ref, idx_ref):
    name = name_ref[...]
    typ = type_ref[...]
    w = w_ref[...]
    dn = (((1,), (1,)), ((), ()))
    a = lax.dot_general(name, w[:, :EMB_D], dn,
                        preferred_element_type=jnp.float32)
    c = lax.dot_general(typ, w[:, EMB_D:], dn,
                        preferred_element_type=jnp.float32)
    s = a[:, None, :] + c[None, :, :] + b_ref[...][None, None, :]
    table_ref[...] = jax.nn.sigmoid(s).reshape(N_COMB, OUT_D)
    f = feat_ref[...]
    idx_ref[...] = f[0] * TYPE_ROWS + f[1]


def _build_table_and_idx(features, name_table, type_table, W, b):
    return pl.pallas_call(
        _tc_body,
        out_shape=(
            jax.ShapeDtypeStruct((N_COMB, OUT_D), jnp.float32),
            jax.ShapeDtypeStruct((B,), jnp.int32),
        ),
    )(features, name_table, type_table, W, b)


def _gather_body(idx_hbm, table_hbm, out_hbm, idx_v, rows_v, sem_g):
    wid = lax.axis_index("s") * _NC + lax.axis_index("c")
    base = wid * _BPW
    for k in range(_NCHUNK):
        pltpu.sync_copy(idx_hbm.at[pl.ds(base + k * _CHUNK, _CHUNK)],
                        idx_v.at[k])
    gathers = [
        pltpu.async_copy(table_hbm.at[idx_v.at[k]],
                         rows_v.at[pl.ds(k * _CHUNK, _CHUNK)], sem_g)
        for k in range(_NCHUNK)
    ]
    for cp in gathers:
        cp.wait()
    pltpu.sync_copy(rows_v, out_hbm.at[pl.ds(base, _BPW)])


def _sc_gather():
    return functools.partial(
        pl.kernel,
        out_type=jax.ShapeDtypeStruct((B, OUT_D), jnp.float32),
        mesh=plsc.VectorSubcoreMesh(core_axis_name="c", subcore_axis_name="s"),
        scratch_types=[
            pltpu.VMEM((_NCHUNK, _CHUNK), jnp.int32),
            pltpu.VMEM((_BPW, OUT_D), jnp.float32),
            pltpu.SemaphoreType.DMA,
        ],
    )(_gather_body)


@jax.jit
def kernel(features, name_table, type_table, W, b):
    feats = features.astype(jnp.int32)
    table, idx = _build_table_and_idx(feats, name_table, type_table, W, b)
    return _sc_gather()(idx, table)


# speedup vs baseline: 1.0212x; 1.0212x over previous
import functools

import jax
import jax.numpy as jnp
from jax import lax
from jax.experimental import pallas as pl
from jax.experimental.pallas import tpu as pltpu
from jax.experimental.pallas import tpu_sc as plsc

B = 16384
OUT_D = 128
NAME_ROWS = 10
TYPE_ROWS = 8
EMB_D = 5
N_COMB = NAME_ROWS * TYPE_ROWS

_NC = 2
_NS = 16
_NW = _NC * _NS
_BPW = B // _NW
_CHUNK = 128
_NCHUNK = _BPW // _CHUNK


def _tc_body(feat_ref, name_ref, type_ref, w_ref, b_ref, table_ref, idx_ref):
    name = name_ref[...]
    typ = type_ref[...]
    w = w_ref[...]
    dn = (((1,), (1,)), ((), ()))
    a = lax.dot_general(name, w[:, :EMB_D], dn,
                        preferred_element_type=jnp.float32)
    c = lax.dot_general(typ, w[:, EMB_D:], dn,
                        preferred_element_type=jnp.float32)
    s = a[:, None, :] + c[None, :, :] + b_ref[...][None, None, :]
    table_ref[...] = jax.nn.sigmoid(s).reshape(N_COMB, OUT_D)
    f = feat_ref[...]
    idx_ref[...] = f[0] * TYPE_ROWS + f[1]


def _build_table_and_idx(features, name_table, type_table, W, b):
    return pl.pallas_call(
        _tc_body,
        out_shape=(
            jax.ShapeDtypeStruct((N_COMB, OUT_D), jnp.float32),
            jax.ShapeDtypeStruct((B,), jnp.int32),
        ),
    )(features, name_table, type_table, W, b)


def _gather_body(idx_hbm, table_hbm, out_hbm, idx_v, rows_v, sem_g):
    wid = lax.axis_index("s") * _NC + lax.axis_index("c")
    base = wid * _BPW
    stages = [
        pltpu.async_copy(idx_hbm.at[pl.ds(base + k * _CHUNK, _CHUNK)],
                         idx_v.at[k], sem_g)
        for k in range(_NCHUNK)
    ]
    for cp in stages:
        cp.wait()
    gathers = [
        pltpu.async_copy(table_hbm.at[idx_v.at[k]],
                         rows_v.at[pl.ds(k * _CHUNK, _CHUNK)], sem_g)
        for k in range(_NCHUNK)
    ]
    for cp in gathers:
        cp.wait()
    pltpu.sync_copy(rows_v, out_hbm.at[pl.ds(base, _BPW)])


def _sc_gather():
    return functools.partial(
        pl.kernel,
        out_type=jax.ShapeDtypeStruct((B, OUT_D), jnp.float32),
        mesh=plsc.VectorSubcoreMesh(core_axis_name="c", subcore_axis_name="s"),
        scratch_types=[
            pltpu.VMEM((_NCHUNK, _CHUNK), jnp.int32),
            pltpu.VMEM((_BPW, OUT_D), jnp.float32),
            pltpu.SemaphoreType.DMA,
        ],
    )(_gather_body)


@jax.jit
def kernel(features, name_table, type_table, W, b):
    feats = features.astype(jnp.int32)
    table, idx = _build_table_and_idx(feats, name_table, type_table, W, b)
    return _sc_gather()(idx, table)
